# Initial kernel scaffold; baseline (speedup 1.0000x reference)
import functools

import jax
import jax.numpy as jnp
from jax import lax
from jax.experimental import pallas as pl
from jax.experimental.pallas import tpu as pltpu
from jax.experimental.pallas import tpu_sc as plsc

B = 16384
L = 20
D = 32
NC = 2
NS = 16
NW = NC * NS
PER_W = B // NW
C = 64
NCHUNK = PER_W // C
IDX_PER_CHUNK = C * L
GSZ = 128
KD = IDX_PER_CHUNK // GSZ
NGROUP = C // 16


def _embed_body(idx_hbm, w_hbm, table_hbm, out_hbm, idx_v, rows_v, w_v, out_v, sem):
    cid = lax.axis_index("c")
    sid = lax.axis_index("s")
    wid = cid * NS + sid

    lane = lax.iota(jnp.int32, 16)

    def chunk_body(ci, carry):
        base = wid * PER_W + ci * C
        pltpu.sync_copy(idx_hbm.at[wid, ci], idx_v)
        copies = []
        for j in range(KD):
            copies.append(
                pltpu.async_copy(
                    table_hbm.at[idx_v.at[j]],
                    rows_v.at[pl.ds(j * GSZ, GSZ)],
                    sem,
                )
            )
        pltpu.sync_copy(w_hbm.at[wid, ci], w_v)
        for cpy in copies:
            cpy.wait()

        for g in range(NGROUP):
            row0 = lane * L + (g * 16 * L)
            brow = lane + (g * 16)

            def l_body(l, accs, row0=row0, g=g):
                wl = w_v[l, pl.ds(g * 16, 16)]
                ridx = row0 + l
                new = []
                for d in range(D):
                    cidx = jnp.full((16,), d, jnp.int32)
                    vals = plsc.load_gather(rows_v, [ridx, cidx])
                    new.append(accs[d] + wl * vals)
                return tuple(new)

            accs = tuple(jnp.zeros((16,), jnp.float32) for _ in range(D))
            accs = lax.fori_loop(0, L, l_body, accs)
            for d in range(D):
                cidx = jnp.full((16,), d, jnp.int32)
                plsc.store_scatter(out_v, [brow, cidx], accs[d])

        pltpu.sync_copy(out_v, out_hbm.at[pl.ds(base, C)])
        return carry

    lax.fori_loop(0, NCHUNK, chunk_body, 0)


_embed_call = functools.partial(
    pl.kernel,
    mesh=plsc.VectorSubcoreMesh(core_axis_name="c", subcore_axis_name="s"),
    out_type=jax.ShapeDtypeStruct((B, D), jnp.float32),
    scratch_types=[
        pltpu.VMEM((KD, GSZ), jnp.int32),
        pltpu.VMEM((IDX_PER_CHUNK, D), jnp.float32),
        pltpu.VMEM((L, C), jnp.float32),
        pltpu.VMEM((C, D), jnp.float32),
        pltpu.SemaphoreType.DMA,
    ],
)(_embed_body)


@jax.jit
def kernel(indices, weights, table):
    idx = indices.astype(jnp.int32).reshape(NW, NCHUNK, KD, GSZ)
    w = (
        weights.astype(jnp.float32)
        .reshape(NW, NCHUNK, C, L)
        .transpose(0, 1, 3, 2)
    )
    return _embed_call(idx, w, table)


# speedup vs baseline: 1.3608x; 1.3608x over previous
import functools

import jax
import jax.numpy as jnp
from jax import lax
from jax.experimental import pallas as pl
from jax.experimental.pallas import tpu as pltpu
from jax.experimental.pallas import tpu_sc as plsc

B = 16384
L = 20
D = 32
NC = 2
NS = 16
NW = NC * NS
PER_W = B // NW
C = 64
NCHUNK = PER_W // C
IDX_PER_CHUNK = C * L
GSZ = 128
KD = IDX_PER_CHUNK // GSZ
NGROUP = C // 16


def _embed_body(idx_hbm, w_hbm, table_hbm, out_hbm, idx_v, rows_v, w_v, out_v, sem):
    cid = lax.axis_index("c")
    sid = lax.axis_index("s")
    wid = cid * NS + sid

    lane = lax.iota(jnp.int32, 16)

    def chunk_body(ci, carry):
        base = wid * PER_W + ci * C
        pltpu.sync_copy(idx_hbm.at[wid, ci], idx_v)
        copies = []
        for j in range(KD):
            copies.append(
                pltpu.async_copy(
                    table_hbm.at[idx_v.at[j]],
                    rows_v.at[pl.ds(j * GSZ, GSZ)],
                    sem,
                )
            )
        pltpu.sync_copy(w_hbm.at[wid, ci], w_v)
        for cpy in copies:
            cpy.wait()

        for g in range(NGROUP):
            row0 = lane * L + (g * 16 * L)
            brow = lane + (g * 16)

            def l_body(l, accs, row0=row0, g=g):
                wl = w_v[l, pl.ds(g * 16, 16)]
                ridx = row0 + l
                new = []
                for d in range(D):
                    cidx = jnp.full((16,), d, jnp.int32)
                    vals = plsc.load_gather(rows_v, [ridx, cidx])
                    new.append(accs[d] + wl * vals)
                return tuple(new)

            accs = tuple(jnp.zeros((16,), jnp.float32) for _ in range(D))
            accs = lax.fori_loop(0, L, l_body, accs)
            for d in range(D):
                cidx = jnp.full((16,), d, jnp.int32)
                plsc.store_scatter(out_v, [brow, cidx], accs[d])

        pltpu.sync_copy(out_v, out_hbm.at[pl.ds(base, C)])
        return carry

    lax.fori_loop(0, NCHUNK, chunk_body, 0)


_embed_call = functools.partial(
    pl.kernel,
    mesh=plsc.VectorSubcoreMesh(core_axis_name="c", subcore_axis_name="s"),
    compiler_params=pltpu.CompilerParams(
        needs_layout_passes=False, use_tc_tiling_on_sc=False
    ),
    out_type=jax.ShapeDtypeStruct((B, D), jnp.float32),
    scratch_types=[
        pltpu.VMEM((KD, GSZ), jnp.int32),
        pltpu.VMEM((IDX_PER_CHUNK, D), jnp.float32),
        pltpu.VMEM((L, C), jnp.float32),
        pltpu.VMEM((C, D), jnp.float32),
        pltpu.SemaphoreType.DMA,
    ],
)(_embed_body)


@jax.jit
def kernel(indices, weights, table):
    idx = indices.astype(jnp.int32).reshape(NW, NCHUNK, KD, GSZ)
    w = (
        weights.astype(jnp.float32)
        .reshape(NW, NCHUNK, C, L)
        .transpose(0, 1, 3, 2)
    )
    return _embed_call(idx, w, table)


# speedup vs baseline: 1.3807x; 1.0146x over previous
import functools

import jax
import jax.numpy as jnp
from jax import lax
from jax.experimental import pallas as pl
from jax.experimental.pallas import tpu as pltpu
from jax.experimental.pallas import tpu_sc as plsc

B = 16384
L = 20
D = 32
NC = 2
NS = 16
NW = NC * NS
PER_W = B // NW
C = 64
NCHUNK = PER_W // C
IDX_PER_CHUNK = C * L
GSZ = 128
KD = IDX_PER_CHUNK // GSZ
NGROUP = C // 16


def _embed_body(
    idx_hbm, w_hbm, table_hbm, out_hbm,
    idx_v0, idx_v1, rows_v0, rows_v1, w_v0, w_v1, out_v, sem0, sem1,
):
    cid = lax.axis_index("c")
    sid = lax.axis_index("s")
    wid = cid * NS + sid

    idx_b = (idx_v0, idx_v1)
    rows_b = (rows_v0, rows_v1)
    w_b = (w_v0, w_v1)
    sems = (sem0, sem1)

    lane = lax.iota(jnp.int32, 16)

    def stage(ci, b):
        pltpu.sync_copy(idx_hbm.at[wid, ci], idx_b[b])
        for j in range(KD):
            pltpu.async_copy(
                table_hbm.at[idx_b[b].at[j]],
                rows_b[b].at[pl.ds(j * GSZ, GSZ)],
                sems[b],
            )
        pltpu.sync_copy(w_hbm.at[wid, ci], w_b[b])

    def drain(b):
        for j in range(KD):
            pltpu.make_async_copy(
                table_hbm.at[idx_b[b].at[j]],
                rows_b[b].at[pl.ds(j * GSZ, GSZ)],
                sems[b],
            ).wait()

    def combine(ci, b):
        rows_v = rows_b[b]
        w_v = w_b[b]
        for g in range(NGROUP):
            row0 = lane * L + (g * 16 * L)
            brow = lane + (g * 16)

            def l_body(l, accs, row0=row0, brow=brow):
                lsplat = jnp.full((16,), 0, jnp.int32) + l
                wl = plsc.load_gather(w_v, [brow, lsplat])
                ridx = row0 + l
                new = []
                for d in range(D):
                    cidx = jnp.full((16,), d, jnp.int32)
                    vals = plsc.load_gather(rows_v, [ridx, cidx])
                    new.append(accs[d] + wl * vals)
                return tuple(new)

            accs = tuple(jnp.zeros((16,), jnp.float32) for _ in range(D))
            accs = lax.fori_loop(0, L, l_body, accs)
            for d in range(D):
                cidx = jnp.full((16,), d, jnp.int32)
                plsc.store_scatter(out_v, [brow, cidx], accs[d])

        base = wid * PER_W + ci * C
        pltpu.sync_copy(out_v, out_hbm.at[pl.ds(base, C)])

    stage(0, 0)

    def outer(c2, carry):
        for b in range(2):
            ci = c2 * 2 + b

            @pl.when(ci + 1 < NCHUNK)
            def _():
                stage(ci + 1, 1 - b)

            drain(b)
            combine(ci, b)
        return carry

    lax.fori_loop(0, NCHUNK // 2, outer, 0)


_embed_call = functools.partial(
    pl.kernel,
    mesh=plsc.VectorSubcoreMesh(core_axis_name="c", subcore_axis_name="s"),
    compiler_params=pltpu.CompilerParams(
        needs_layout_passes=False, use_tc_tiling_on_sc=False
    ),
    out_type=jax.ShapeDtypeStruct((B, D), jnp.float32),
    scratch_types=[
        pltpu.VMEM((KD, GSZ), jnp.int32),
        pltpu.VMEM((KD, GSZ), jnp.int32),
        pltpu.VMEM((IDX_PER_CHUNK, D), jnp.float32),
        pltpu.VMEM((IDX_PER_CHUNK, D), jnp.float32),
        pltpu.VMEM((C, L), jnp.float32),
        pltpu.VMEM((C, L), jnp.float32),
        pltpu.VMEM((C, D), jnp.float32),
        pltpu.SemaphoreType.DMA,
        pltpu.SemaphoreType.DMA,
    ],
)(_embed_body)


@jax.jit
def kernel(indices, weights, table):
    idx = indices.astype(jnp.int32).reshape(NW, NCHUNK, KD, GSZ)
    w = weights.astype(jnp.float32).reshape(NW, NCHUNK, C, L)
    return _embed_call(idx, w, table)


# speedup vs baseline: 1.5121x; 1.0952x over previous
import functools

import jax
import jax.numpy as jnp
from jax import lax
from jax.experimental import pallas as pl
from jax.experimental.pallas import tpu as pltpu
from jax.experimental.pallas import tpu_sc as plsc

B = 16384
L = 20
D = 32
V = 1000000
NC = 2
NS = 16
NW = NC * NS
PER_W = B // NW
C = 16
NCHUNK = PER_W // C
IDX_PER_CHUNK = C * L
SPAD = 32
RPS = 128 // D
OROWS = 8

lane16 = None


def _embed_body(
    idx_hbm, w_hbm, rcol_hbm, table_hbm, out_hbm,
    idx_v0, idx_v1, rows_v0, rows_v1, w_v0, w_v1, rc_v0, rc_v1,
    out_v, sem0, sem1,
):
    cid = lax.axis_index("c")
    sid = lax.axis_index("s")
    wid = cid * NS + sid

    idx_b = (idx_v0, idx_v1)
    rows_b = (rows_v0, rows_v1)
    w_b = (w_v0, w_v1)
    rc_b = (rc_v0, rc_v1)
    sems = (sem0, sem1)

    lane = lax.iota(jnp.int32, 16)

    g_slices = [(0, 128), (128, 128), (256, 64)]

    def stage(ci, b):
        pltpu.sync_copy(idx_hbm.at[wid, ci, 0], idx_b[b])
        for s0, sn in g_slices:
            pltpu.async_copy(
                table_hbm.at[idx_b[b].at[pl.ds(s0, sn)]],
                rows_b[b].at[pl.ds(s0, sn)],
                sems[b],
            )
        pltpu.sync_copy(w_hbm.at[wid, ci, 0], w_b[b])
        pltpu.sync_copy(rcol_hbm.at[wid, ci, 0], rc_b[b])

    def drain(b):
        for s0, sn in g_slices:
            pltpu.make_async_copy(
                table_hbm.at[idx_b[b].at[pl.ds(s0, sn)]],
                rows_b[b].at[pl.ds(s0, sn)],
                sems[b],
            ).wait()

    def combine(ci, b):
        rows_v = rows_b[b]
        w_v = w_b[b]
        rc_v = rc_b[b]

        def b_body(bb, carry):
            woff = pl.multiple_of(bb * SPAD, SPAD)
            w0 = w_v[pl.ds(woff, 16)]
            w1 = w_v[pl.ds(woff + 16, 16)]
            rc0 = rc_v[pl.ds(woff, 16)]
            rc1 = rc_v[pl.ds(woff + 16, 16)]
            acc = [jnp.zeros((16,), jnp.float32) for _ in range(4)]
            slot0 = bb * L
            for l in range(L):
                ws, rs, j = (w0, rc0, l) if l < 16 else (w1, rc1, l - 16)
                cidx = jnp.full((16,), j, jnp.int32)
                wl = ws.at[cidx].get(mode="promise_in_bounds")
                rc = rs.at[cidx].get(mode="promise_in_bounds")
                rsplat = jnp.full((16,), 0, jnp.int32) + (slot0 + l)
                col = rc + lane
                g0 = plsc.load_gather(rows_v, [rsplat, col])
                g1 = plsc.load_gather(rows_v, [rsplat, col + 16])
                p = 2 * (l % 2)
                acc[p] = acc[p] + wl * g0
                acc[p + 1] = acc[p + 1] + wl * g1
            obase = (ci % 2) * C + bb
            orow = jnp.full((16,), 0, jnp.int32) + (obase // RPS)
            ocol0 = (obase % RPS) * D
            plsc.store_scatter(out_v, [orow, ocol0 + lane], acc[0] + acc[2])
            plsc.store_scatter(out_v, [orow, ocol0 + 16 + lane], acc[1] + acc[3])
            return carry

        lax.fori_loop(0, C, b_body, 0)

        @pl.when(ci % 2 == 1)
        def _():
            base = pl.multiple_of(
                wid * (PER_W * D // 128) + (ci // 2) * OROWS, OROWS
            )
            pltpu.sync_copy(out_v, out_hbm.at[pl.ds(base, OROWS)])

    stage(0, 0)

    def outer(c2, carry):
        for b in range(2):
            ci = c2 * 2 + b

            @pl.when(ci + 1 < NCHUNK)
            def _():
                stage(ci + 1, 1 - b)

            drain(b)
            combine(ci, b)
        return carry

    lax.fori_loop(0, NCHUNK // 2, outer, 0)


_embed_call = functools.partial(
    pl.kernel,
    mesh=plsc.VectorSubcoreMesh(core_axis_name="c", subcore_axis_name="s"),
    compiler_params=pltpu.CompilerParams(
        needs_layout_passes=False, use_tc_tiling_on_sc=True
    ),
    out_type=jax.ShapeDtypeStruct((B * D // 128, 128), jnp.float32),
    scratch_types=[
        pltpu.VMEM((IDX_PER_CHUNK,), jnp.int32),
        pltpu.VMEM((IDX_PER_CHUNK,), jnp.int32),
        pltpu.VMEM((IDX_PER_CHUNK, 128), jnp.float32),
        pltpu.VMEM((IDX_PER_CHUNK, 128), jnp.float32),
        pltpu.VMEM((C * SPAD,), jnp.float32),
        pltpu.VMEM((C * SPAD,), jnp.float32),
        pltpu.VMEM((C * SPAD,), jnp.int32),
        pltpu.VMEM((C * SPAD,), jnp.int32),
        pltpu.VMEM((OROWS, 128), jnp.float32),
        pltpu.SemaphoreType.DMA,
        pltpu.SemaphoreType.DMA,
    ],
)(_embed_body)


@jax.jit
def kernel(indices, weights, table):
    idx32 = indices.astype(jnp.int32)
    q = (idx32 // RPS).reshape(NW, NCHUNK, 1, IDX_PER_CHUNK)
    rc = jnp.pad((idx32 % RPS) * D, ((0, 0), (0, SPAD - L)))
    rc = rc.reshape(NW, NCHUNK, 1, C * SPAD)
    w = jnp.pad(weights.astype(jnp.float32), ((0, 0), (0, SPAD - L)))
    w = w.reshape(NW, NCHUNK, 1, C * SPAD)
    tb = table.reshape(V // RPS, 128)
    out = _embed_call(q, w, rc, tb)
    return out.reshape(B, D)
